# baseline (speedup 1.0000x reference)
import functools

import jax
import jax.numpy as jnp
from jax.experimental import pallas as pl
from jax.experimental.pallas import tpu as pltpu

NUM_EXPERTS = 8
TOPK = 2
HIDDEN = 1024
T_BLK = 1024
NBUF = 6


def _router_body(nblk, w_ref, h_ref, probs_ref, idx_ref, *scratch):
    bufs = scratch[:NBUF]
    sems = scratch[NBUF:]
    w = w_ref[...]

    def mk(b, slot):
        return pltpu.make_async_copy(
            h_ref.at[pl.ds(b * T_BLK, T_BLK), :],
            bufs[slot],
            sems[slot],
        )

    for b in range(min(NBUF, nblk)):
        mk(b, b).start(priority=b % 2)

    for b in range(nblk):
        slot = b % NBUF
        mk(b, slot).wait()
        h = bufs[slot][...]
        logits = jax.lax.dot_general(
            h, w, (((1,), (1,)), ((), ())),
            preferred_element_type=jnp.float32,
        )

        e_iota = jax.lax.broadcasted_iota(jnp.int32, logits.shape, 1)
        m1 = jnp.max(logits, axis=-1)
        i1 = jnp.argmax(logits, axis=-1).astype(jnp.int32)
        masked = jnp.where(e_iota == i1[:, None], -jnp.inf, logits)
        m2 = jnp.max(masked, axis=-1)
        i2 = jnp.argmax(masked, axis=-1).astype(jnp.int32)

        ed = jnp.exp(m2 - m1)
        denom = 1.0 + ed
        p1 = 1.0 / denom
        p2 = ed / denom

        probs_ref[pl.ds(b * T_BLK, T_BLK), :] = jnp.stack([p1, p2], axis=-1)
        idx_ref[pl.ds(b * T_BLK, T_BLK), :] = jnp.stack([i1, i2], axis=-1)

        if b + NBUF < nblk:
            mk(b + NBUF, slot).start(priority=slot % 2)


@jax.jit
def kernel(hidden_states, weight):
    S, B, H = hidden_states.shape
    T = S * B
    nblk = T // T_BLK
    h = hidden_states.reshape(T, H)
    probs, idx = pl.pallas_call(
        functools.partial(_router_body, nblk),
        in_specs=[
            pl.BlockSpec(memory_space=pltpu.MemorySpace.VMEM),
            pl.BlockSpec(memory_space=pl.ANY),
        ],
        out_specs=[
            pl.BlockSpec(memory_space=pltpu.MemorySpace.VMEM),
            pl.BlockSpec(memory_space=pltpu.MemorySpace.VMEM),
        ],
        out_shape=[
            jax.ShapeDtypeStruct((T, TOPK), jnp.float32),
            jax.ShapeDtypeStruct((T, TOPK), jnp.int32),
        ],
        scratch_shapes=(
            [pltpu.VMEM((T_BLK, HIDDEN), jnp.float32) for _ in range(NBUF)]
            + [pltpu.SemaphoreType.DMA for _ in range(NBUF)]
        ),
    )(weight, h)
    return (probs, idx)


# speedup vs baseline: 2.5958x; 2.5958x over previous
import functools

import jax
import jax.numpy as jnp
from jax.experimental import pallas as pl
from jax.experimental.pallas import tpu as pltpu

NUM_EXPERTS = 8
TOPK = 2
HIDDEN = 1024
T_BLK = 1024
NBUF = 6


def _router_body(nblk, w_ref, h_ref, probs_ref, idx_ref, *scratch):
    bufs = scratch[:NBUF]
    sems = scratch[NBUF:]
    w = w_ref[...]
    h2 = h_ref.reshape(nblk * T_BLK, HIDDEN)

    def mk(b, slot):
        return pltpu.make_async_copy(
            h2.at[pl.ds(b * T_BLK, T_BLK), :],
            bufs[slot],
            sems[slot],
        )

    for b in range(min(NBUF, nblk)):
        mk(b, b).start()

    for b in range(nblk):
        slot = b % NBUF
        mk(b, slot).wait()
        h = bufs[slot][...]
        logits = jax.lax.dot_general(
            h, w, (((1,), (1,)), ((), ())),
            preferred_element_type=jnp.float32,
        )

        e_iota = jax.lax.broadcasted_iota(jnp.int32, logits.shape, 1)
        m1 = jnp.max(logits, axis=-1)
        i1 = jnp.argmax(logits, axis=-1).astype(jnp.int32)
        masked = jnp.where(e_iota == i1[:, None], -jnp.inf, logits)
        m2 = jnp.max(masked, axis=-1)
        i2 = jnp.argmax(masked, axis=-1).astype(jnp.int32)

        ed = jnp.exp(m2 - m1)
        denom = 1.0 + ed
        p1 = 1.0 / denom
        p2 = ed / denom

        probs_ref[pl.ds(b * T_BLK, T_BLK), :] = jnp.stack([p1, p2], axis=-1)
        idx_ref[pl.ds(b * T_BLK, T_BLK), :] = jnp.stack([i1, i2], axis=-1)

        if b + NBUF < nblk:
            mk(b + NBUF, slot).start()


@jax.jit
def kernel(hidden_states, weight):
    S, B, H = hidden_states.shape
    T = S * B
    nblk = T // T_BLK
    probs, idx = pl.pallas_call(
        functools.partial(_router_body, nblk),
        in_specs=[
            pl.BlockSpec(memory_space=pltpu.MemorySpace.VMEM),
            pl.BlockSpec(memory_space=pl.ANY),
        ],
        out_specs=[
            pl.BlockSpec(memory_space=pltpu.MemorySpace.VMEM),
            pl.BlockSpec(memory_space=pltpu.MemorySpace.VMEM),
        ],
        out_shape=[
            jax.ShapeDtypeStruct((T, TOPK), jnp.float32),
            jax.ShapeDtypeStruct((T, TOPK), jnp.int32),
        ],
        scratch_shapes=(
            [pltpu.VMEM((T_BLK, HIDDEN), jnp.float32) for _ in range(NBUF)]
            + [pltpu.SemaphoreType.DMA for _ in range(NBUF)]
        ),
    )(weight, hidden_states)
    return (probs, idx)
